# baseline (speedup 1.0000x reference)
import functools

import jax
import jax.numpy as jnp
from jax import lax
from jax.experimental import pallas as pl
from jax.experimental.pallas import tpu as pltpu
from jax.experimental.pallas import tpu_sc as plsc

_LANES = 16
_CHUNK = 128


def _tc_body(y_ref, x_ref, o_ref):
    x = x_ref[...]
    y = y_ref[...]
    shape3 = x.shape
    y3 = jax.lax.broadcast_in_dim(y, shape3, (0, 1))
    mask3 = y3 != -1
    safe3 = jnp.where(mask3, y3, 0)
    iota3 = jax.lax.broadcasted_iota(jnp.int32, shape3, 2)
    g = jnp.sum(jnp.where(iota3 == safe3, x, 0.0), axis=-1)
    lse = jnp.log(jnp.sum(jnp.exp(x), axis=-1))
    loss = jnp.where(y != -1, lse - g, 0.0)
    o_ref[0] = jnp.sum(loss, axis=-1, keepdims=True)


def _sc_body(rows_w, row_tc0, c, x_hbm, x2d_hbm, y_hbm, s_hbm, g_hbm,
             xbuf, ylab, idx_v, sbuf, chunks, sem):
    nc = 2
    wid = lax.axis_index("s") * nc + lax.axis_index("c")
    row0 = row_tc0 + wid * rows_w
    pltpu.sync_copy(y_hbm.at[pl.ds(row0, rows_w)], ylab)
    pltpu.sync_copy(x_hbm.at[pl.ds(row0 * c, rows_w * c)], xbuf)

    iota16 = lax.broadcasted_iota(jnp.int32, (_LANES,), 0)
    for grp in range(rows_w // _LANES):
        rowv = row0 + grp * _LANES + iota16
        lab = ylab[pl.ds(grp * _LANES, _LANES)]
        safe = jnp.maximum(lab, 0)
        flat = rowv * c + safe
        idx_v[pl.ds(grp * _LANES, _LANES)] = lax.shift_right_logical(flat, 7)

    gather = pltpu.async_copy(x2d_hbm.at[idx_v], chunks, sem)

    def row_body(r, carry):
        acc = jnp.zeros((_LANES,), jnp.float32)
        base = r * c
        for j in range(c // _LANES):
            acc = acc + jnp.exp(xbuf[pl.ds(base + j * _LANES, _LANES)])
        sbuf[pl.ds(r * _LANES, _LANES)] = acc
        return carry

    lax.fori_loop(0, rows_w, row_body, 0)
    gather.wait()

    pltpu.sync_copy(sbuf, s_hbm.at[pl.ds(wid * rows_w * _LANES,
                                         rows_w * _LANES)])
    pltpu.sync_copy(chunks, g_hbm.at[pl.ds(wid * rows_w, rows_w)])


def _combine_body(y_ref, s_ref, g_ref, o_ref):
    s = jnp.sum(s_ref[...], axis=-1)
    y = y_ref[...]
    off = jnp.maximum(y, 0) & (_CHUNK - 1)
    iota = lax.broadcasted_iota(jnp.int32, g_ref.shape, 2)
    g = jnp.sum(jnp.where(iota == off[..., None], g_ref[...], 0.0), axis=-1)
    loss = jnp.where(y != -1, jnp.log(s) - g, 0.0)
    o_ref[...] = jnp.sum(loss, axis=-1, keepdims=True)


def kernel(y_true, y_pred):
    b, a, c = y_pred.shape
    b_sc = 192
    b_tc = b - b_sc
    bbb = 64
    nblk = b_tc // bbb
    nw = 32
    rows_sc = b_sc * a
    rows_w = rows_sc // nw
    row_tc0 = b_tc * a

    yi = y_true.astype(jnp.int32)

    tc_out = pl.pallas_call(
        _tc_body,
        grid=(nblk,),
        in_specs=[
            pl.BlockSpec((bbb, a), lambda i: (i, 0)),
            pl.BlockSpec((bbb, a, c), lambda i: (i, 0, 0)),
        ],
        out_specs=pl.BlockSpec((1, bbb, 1), lambda i: (i, 0, 0)),
        out_shape=jax.ShapeDtypeStruct((nblk, bbb, 1), jnp.float32),
    )(yi, y_pred)

    x_flat = y_pred.reshape(b * a * c)
    x_2d = y_pred.reshape(b * a * c // _CHUNK, _CHUNK)
    y_rows = yi.reshape(b * a)
    mesh = plsc.VectorSubcoreMesh(core_axis_name="c", subcore_axis_name="s")
    s_part, g_chunks = pl.kernel(
        functools.partial(_sc_body, rows_w, row_tc0, c),
        out_type=[
            jax.ShapeDtypeStruct((rows_sc * _LANES,), jnp.float32),
            jax.ShapeDtypeStruct((rows_sc, _CHUNK), jnp.float32),
        ],
        mesh=mesh,
        scratch_types=[
            pltpu.VMEM((rows_w * c,), jnp.float32),
            pltpu.VMEM((rows_w,), jnp.int32),
            pltpu.VMEM((rows_w,), jnp.int32),
            pltpu.VMEM((rows_w * _LANES,), jnp.float32),
            pltpu.VMEM((rows_w, _CHUNK), jnp.float32),
            pltpu.SemaphoreType.DMA,
        ],
    )(x_flat, x_2d, y_rows)

    sc_out = pl.pallas_call(
        _combine_body,
        in_specs=[
            pl.BlockSpec((b_sc, a), lambda: (0, 0)),
            pl.BlockSpec((b_sc, a, _LANES), lambda: (0, 0, 0)),
            pl.BlockSpec((b_sc, a, _CHUNK), lambda: (0, 0, 0)),
        ],
        out_specs=pl.BlockSpec((b_sc, 1), lambda: (0, 0)),
        out_shape=jax.ShapeDtypeStruct((b_sc, 1), jnp.float32),
    )(yi[b_tc:], s_part.reshape(b_sc, a, _LANES),
      g_chunks.reshape(b_sc, a, _CHUNK))

    return jnp.concatenate([tc_out.reshape(b_tc), sc_out.reshape(b_sc)])


# speedup vs baseline: 2.5645x; 2.5645x over previous
import functools

import jax
import jax.numpy as jnp
from jax import lax
from jax.experimental import pallas as pl
from jax.experimental.pallas import tpu as pltpu
from jax.experimental.pallas import tpu_sc as plsc

_LANES = 16


def _tc_body(y_ref, x_ref, o_ref):
    x = x_ref[...]
    y = y_ref[...]
    shape3 = x.shape
    y3 = jax.lax.broadcast_in_dim(y, shape3, (0, 1))
    mask3 = y3 != -1
    safe3 = jnp.where(mask3, y3, 0)
    iota3 = jax.lax.broadcasted_iota(jnp.int32, shape3, 2)
    g = jnp.sum(jnp.where(iota3 == safe3, x, 0.0), axis=-1)
    lse = jnp.log(jnp.sum(jnp.exp(x), axis=-1))
    loss = jnp.where(y != -1, lse - g, 0.0)
    o_ref[0] = jnp.sum(loss, axis=-1, keepdims=True)


def _sc_body(rows_w, row_tc0, c, x_hbm, y_hbm, s_hbm, g_hbm,
             xbuf, ylab, sbuf, gbuf):
    nc = 2
    wid = lax.axis_index("s") * nc + lax.axis_index("c")
    row0 = row_tc0 + wid * rows_w
    pltpu.sync_copy(y_hbm.at[pl.ds(row0, rows_w)], ylab)
    pltpu.sync_copy(x_hbm.at[pl.ds(row0, rows_w)], xbuf)

    iota16 = lax.broadcasted_iota(jnp.int32, (_LANES,), 0)

    def row_body(r, carry):
        labv = ylab[r, pl.ds(0, _LANES)]
        acc = jnp.zeros((_LANES,), jnp.float32)
        gacc = jnp.zeros((_LANES,), jnp.float32)
        for j in range(c // _LANES):
            v = xbuf[r, pl.ds(j * _LANES, _LANES)]
            acc = acc + jnp.exp(v)
            gacc = gacc + jnp.where(iota16 == labv, v, 0.0)
            labv = labv - _LANES
        sbuf[pl.ds(r * _LANES, _LANES)] = acc
        gbuf[pl.ds(r * _LANES, _LANES)] = gacc
        return carry

    lax.fori_loop(0, rows_w, row_body, 0)

    pltpu.sync_copy(sbuf, s_hbm.at[pl.ds(wid * rows_w * _LANES,
                                         rows_w * _LANES)])
    pltpu.sync_copy(gbuf, g_hbm.at[pl.ds(wid * rows_w * _LANES,
                                         rows_w * _LANES)])


def _combine_body(y_ref, s_ref, g_ref, o_ref):
    s = jnp.sum(s_ref[...], axis=-1)
    g = jnp.sum(g_ref[...], axis=-1)
    y = y_ref[...]
    loss = jnp.where(y != -1, jnp.log(s) - g, 0.0)
    o_ref[...] = jnp.sum(loss, axis=-1, keepdims=True)


def kernel(y_true, y_pred):
    b, a, c = y_pred.shape
    b_sc = 128
    b_tc = b - b_sc
    bbb = 64
    nblk = b_tc // bbb
    nw = 32
    rows_sc = b_sc * a
    rows_w = rows_sc // nw
    row_tc0 = b_tc * a

    yi = y_true.astype(jnp.int32)

    tc_out = pl.pallas_call(
        _tc_body,
        grid=(nblk,),
        in_specs=[
            pl.BlockSpec((bbb, a), lambda i: (i, 0)),
            pl.BlockSpec((bbb, a, c), lambda i: (i, 0, 0)),
        ],
        out_specs=pl.BlockSpec((1, bbb, 1), lambda i: (i, 0, 0)),
        out_shape=jax.ShapeDtypeStruct((nblk, bbb, 1), jnp.float32),
    )(yi, y_pred)

    x_rows = y_pred.reshape(b * a, c)
    y_splat = jnp.broadcast_to(
        jnp.maximum(yi.reshape(b * a), 0)[:, None], (b * a, _LANES))
    mesh = plsc.VectorSubcoreMesh(core_axis_name="c", subcore_axis_name="s")
    s_part, g_part = pl.kernel(
        functools.partial(_sc_body, rows_w, row_tc0, c),
        out_type=[
            jax.ShapeDtypeStruct((rows_sc * _LANES,), jnp.float32),
            jax.ShapeDtypeStruct((rows_sc * _LANES,), jnp.float32),
        ],
        mesh=mesh,
        scratch_types=[
            pltpu.VMEM((rows_w, c), jnp.float32),
            pltpu.VMEM((rows_w, _LANES), jnp.int32),
            pltpu.VMEM((rows_w * _LANES,), jnp.float32),
            pltpu.VMEM((rows_w * _LANES,), jnp.float32),
        ],
    )(x_rows, y_splat)

    sc_out = pl.pallas_call(
        _combine_body,
        in_specs=[
            pl.BlockSpec((b_sc, a), lambda: (0, 0)),
            pl.BlockSpec((b_sc, a, _LANES), lambda: (0, 0, 0)),
            pl.BlockSpec((b_sc, a, _LANES), lambda: (0, 0, 0)),
        ],
        out_specs=pl.BlockSpec((b_sc, 1), lambda: (0, 0)),
        out_shape=jax.ShapeDtypeStruct((b_sc, 1), jnp.float32),
    )(yi[b_tc:], s_part.reshape(b_sc, a, _LANES),
      g_part.reshape(b_sc, a, _LANES))

    return jnp.concatenate([tc_out.reshape(b_tc), sc_out.reshape(b_sc)])
